# baseline (speedup 1.0000x reference)
import functools

import jax
import jax.numpy as jnp
from jax import lax
from jax.experimental import pallas as pl
from jax.experimental.pallas import tpu as pltpu
from jax.experimental.pallas import tpu_sc as plsc

NUM_FIELDS = 26
VOCAB = 100000
EMB_DIM = 32
OUT_DIM = 128
BATCH = 16384
EPS = 1e-5

NUM_CORES = 2
NUM_SUBCORES = 16
LANES = 16

TROWS = NUM_FIELDS * EMB_DIM
CW = 2048
HW = 1024
NMAIN = 48
LASTW = 1664
TAILLO = NMAIN * CW + LASTW
TAILW = VOCAB - TAILLO
PADB = BATCH + 64
OUTROWS = NUM_FIELDS * PADB
MCAP = 512
MGCAP = 1024
NSUP = 9
SUPCAP = 3072
PIECE = 512
POISON = 0x7FFFFFFF


def _sc_body(xT_hbm, tabT_hbm, tail_hbm, out_hbm,
             idx_v, slab_v, tail_v, mg_v, mg2_v, mv_v, staged_v, sup_v,
             cnt_sm, dsem, ssem):
    wid = lax.axis_index("s") * NUM_CORES + lax.axis_index("c")

    @pl.when(wid < NUM_FIELDS)
    def _worker():
        row0 = pl.multiple_of(wid * EMB_DIM, EMB_DIM)
        dump = wid * PADB + BATCH
        iota = lax.iota(jnp.int32, LANES)

        pltpu.sync_copy(tail_hbm.at[pl.ds(pl.multiple_of(wid * 8, 8), 8)], tail_v)

        def pinit(i, c):
            sup_v[pl.ds(i * LANES, LANES)] = jnp.full((LANES,), POISON, jnp.int32)
            return c

        lax.fori_loop(0, NSUP * SUPCAP // LANES, pinit, 0)

        def piece_body(k, cnts):
            off = pl.multiple_of(wid * BATCH + k * PIECE, PIECE)
            pltpu.sync_copy(xT_hbm.at[pl.ds(off, PIECE)], idx_v)

            def group(i, cnts):
                v16 = idx_v[pl.ds(i * LANES, LANES)]
                p16 = v16 * 16384 + (k * PIECE + i * LANES + iota)
                s16 = ((v16 >> 12) * 21846) >> 16
                new = []
                for s in range(NSUP):
                    m = s16 == s
                    plsc.store_compressed(
                        sup_v.at[pl.ds(s * SUPCAP + cnts[s], LANES)], p16, mask=m)
                    new.append(cnts[s] + plsc.all_reduce_population_count(m)[0])
                return tuple(new)

            return lax.fori_loop(0, PIECE // LANES, group, cnts)

        cnts = lax.fori_loop(0, BATCH // PIECE, piece_body, (0,) * NSUP)
        for s in range(NSUP):
            cnt_sm[s] = cnts[s]

        def scan_list(ns, base, lo, hi):
            def ibody(i, c):
                mg_v[pl.ds(i * LANES, LANES)] = jnp.full((LANES,), dump, jnp.int32)
                return c

            lax.fori_loop(0, MCAP // LANES, ibody, 0)
            groups = jnp.minimum((ns + LANES - 1) // LANES, SUPCAP // LANES)

            def sbody(j, cnt):
                p16 = sup_v[pl.ds(base + j * LANES, LANES)]
                v16 = p16 >> 14
                m = (v16 >= lo) & (v16 < hi)
                dst16 = wid * PADB + (p16 & (16384 - 1))
                plsc.store_compressed(mg_v.at[pl.ds(cnt, LANES)], dst16, mask=m)
                plsc.store_compressed(mv_v.at[pl.ds(cnt, LANES)], v16, mask=m)
                return cnt + plsc.all_reduce_population_count(m)[0]

            return lax.fori_loop(0, groups, sbody, 0)

        def extract_half(n, lo, h):
            groups = jnp.minimum((n + LANES - 1) // LANES, MCAP // LANES)

            def ebody(j, c):
                vl = mv_v[pl.ds(j * LANES, LANES)] - lo
                m = (vl >> 10) == h
                col = vl & (HW - 1)
                r16 = j * LANES + iota
                for e in range(EMB_DIM):
                    e16 = jnp.full((LANES,), e, jnp.int32)
                    vals = plsc.load_gather(slab_v, [e16, col], mask=m)
                    plsc.store_scatter(staged_v, [r16, e16], vals, mask=m)
                return c

            lax.fori_loop(0, groups, ebody, 0)

        def extract_tail(n):
            groups = jnp.minimum((n + LANES - 1) // LANES, MCAP // LANES)

            def ebody(j, c):
                vl = mv_v[pl.ds(j * LANES, LANES)] - TAILLO
                r16 = j * LANES + iota
                for e in range(EMB_DIM):
                    flat = e * TAILW + vl
                    vals = plsc.load_gather(tail_v, [flat >> 7, flat & 127])
                    plsc.store_scatter(
                        staged_v, [r16, jnp.full((LANES,), e, jnp.int32)], vals)
                return c

            lax.fori_loop(0, groups, ebody, 0)

        def flush():
            for j in range(MCAP // 128):
                for k in range(128 // LANES):
                    mg2_v[j, pl.ds(k * LANES, LANES)] = (
                        mg_v[pl.ds(j * 128 + k * LANES, LANES)])
            copies = [
                pltpu.async_copy(staged_v.at[pl.ds(j * 128, 128)],
                                 out_hbm.at[mg2_v.at[j]], ssem)
                for j in range(MCAP // 128)
            ]
            for cp in copies:
                cp.wait()

        def main_chunk(c, carry):
            c6 = c // 6
            lo = pl.multiple_of(c * CW, CW)
            ns = cnt_sm[c6]
            cp = pltpu.async_copy(
                tabT_hbm.at[pl.ds(row0, EMB_DIM), pl.ds(lo, HW)], slab_v, dsem)
            n = scan_list(ns, c6 * SUPCAP, lo, lo + CW)
            cp.wait()
            extract_half(n, lo, 0)
            cp = pltpu.async_copy(
                tabT_hbm.at[pl.ds(row0, EMB_DIM),
                            pl.ds(pl.multiple_of(lo + HW, HW), HW)],
                slab_v, dsem)
            cp.wait()
            extract_half(n, lo, 1)
            flush()
            return carry

        lax.fori_loop(0, NMAIN, main_chunk, 0)

        lo = NMAIN * CW
        cp = pltpu.async_copy(
            tabT_hbm.at[pl.ds(row0, EMB_DIM), pl.ds(lo, HW)], slab_v, dsem)
        n = scan_list(cnt_sm[8], 8 * SUPCAP, lo, TAILLO)
        cp.wait()
        extract_half(n, lo, 0)
        cp = pltpu.async_copy(
            tabT_hbm.at[pl.ds(row0, EMB_DIM), pl.ds(lo + HW, LASTW - HW)],
            slab_v.at[:, pl.ds(0, LASTW - HW)], dsem)
        cp.wait()
        extract_half(n, lo, 1)
        flush()

        n = scan_list(cnt_sm[8], 8 * SUPCAP, TAILLO, VOCAB)
        extract_tail(n)
        flush()


_sc_lookup = functools.partial(
    pl.kernel,
    mesh=plsc.VectorSubcoreMesh(core_axis_name="c", subcore_axis_name="s"),
    out_type=jax.ShapeDtypeStruct((OUTROWS, 128), jnp.float32),
    scratch_types=[
        pltpu.VMEM((PIECE,), jnp.int32),
        pltpu.VMEM((EMB_DIM, HW), jnp.float32),
        pltpu.VMEM((8, 128), jnp.float32),
        pltpu.VMEM((MGCAP,), jnp.int32),
        pltpu.VMEM((MCAP // 128, 128), jnp.int32),
        pltpu.VMEM((MGCAP,), jnp.int32),
        pltpu.VMEM((MCAP, 128), jnp.float32),
        pltpu.VMEM((NSUP * SUPCAP,), jnp.int32),
        pltpu.SMEM((16,), jnp.int32),
        pltpu.SemaphoreType.DMA,
        pltpu.SemaphoreType.DMA,
    ],
    compiler_params=pltpu.CompilerParams(
        use_tc_tiling_on_sc=True, needs_layout_passes=False
    ),
)(_sc_body)


BB = 512


def _tc_proj_body(c_ref, w_ref, b_ref, g_ref, be_ref, o_ref):
    h = jnp.zeros((BB, OUT_DIM), jnp.float32)
    for f in range(NUM_FIELDS):
        h = h + jnp.dot(c_ref[f, :, :EMB_DIM], w_ref[f],
                        preferred_element_type=jnp.float32)
    h = jnp.maximum(h + b_ref[...], 0.0)
    mean = jnp.mean(h, axis=1, keepdims=True)
    cen = h - mean
    var = jnp.mean(cen * cen, axis=1, keepdims=True)
    o_ref[...] = cen * lax.rsqrt(var + EPS) * g_ref[...] + be_ref[...]


def _tc_proj(rows3, W3, b, gamma, beta):
    return pl.pallas_call(
        _tc_proj_body,
        grid=(BATCH // BB,),
        in_specs=[
            pl.BlockSpec((NUM_FIELDS, BB, 128), lambda i: (0, i, 0)),
            pl.BlockSpec((NUM_FIELDS, EMB_DIM, OUT_DIM), lambda i: (0, 0, 0)),
            pl.BlockSpec((1, OUT_DIM), lambda i: (0, 0)),
            pl.BlockSpec((1, OUT_DIM), lambda i: (0, 0)),
            pl.BlockSpec((1, OUT_DIM), lambda i: (0, 0)),
        ],
        out_specs=pl.BlockSpec((BB, OUT_DIM), lambda i: (i, 0)),
        out_shape=jax.ShapeDtypeStruct((BATCH, OUT_DIM), jnp.float32),
        compiler_params=pltpu.CompilerParams(
            dimension_semantics=("arbitrary",),
        ),
    )(rows3, W3, b, gamma, beta)


def kernel(x, tables, W, b, gamma, beta):
    xT_flat = x.astype(jnp.int32).T.reshape(NUM_FIELDS * BATCH)
    tabT = tables.transpose(0, 2, 1).reshape(TROWS, VOCAB)
    tail = tabT[:, TAILLO:VOCAB].reshape(TROWS * TAILW // 128, 128)

    rows = _sc_lookup(xT_flat, tabT, tail)
    rows3 = rows.reshape(NUM_FIELDS, PADB, 128)
    return _tc_proj(
        rows3,
        W.reshape(NUM_FIELDS, EMB_DIM, OUT_DIM),
        b.reshape(1, OUT_DIM),
        gamma.reshape(1, OUT_DIM),
        beta.reshape(1, OUT_DIM),
    )


# speedup vs baseline: 1.0030x; 1.0030x over previous
import functools

import jax
import jax.numpy as jnp
from jax import lax
from jax.experimental import pallas as pl
from jax.experimental.pallas import tpu as pltpu
from jax.experimental.pallas import tpu_sc as plsc

NUM_FIELDS = 26
VOCAB = 100000
EMB_DIM = 32
OUT_DIM = 128
BATCH = 16384
EPS = 1e-5

NUM_CORES = 2
NUM_SUBCORES = 16
LANES = 16

TROWS = NUM_FIELDS * EMB_DIM
CW = 2048
HW = 1024
NMAIN = 48
LASTW = 1664
TAILLO = NMAIN * CW + LASTW
TAILW = VOCAB - TAILLO
PADB = BATCH + 64
OUTROWS = NUM_FIELDS * PADB
MCAP = 512
MGCAP = 1024
NSUP = 9
SUPCAP = 3072
PIECE = 512
POISON = 0x7FFFFFFF


def _sc_body(xT_hbm, tabT_hbm, tail_hbm, out_hbm,
             idx_v, slab_v, tail_v, mg_v, mg2_v, mv_v, staged_v, sup_v,
             cnt_sm, dsem, ssem):
    wid = lax.axis_index("s") * NUM_CORES + lax.axis_index("c")

    @pl.when(wid < NUM_FIELDS)
    def _worker():
        row0 = pl.multiple_of(wid * EMB_DIM, EMB_DIM)
        dump = wid * PADB + BATCH
        iota = lax.iota(jnp.int32, LANES)

        pltpu.sync_copy(tail_hbm.at[pl.ds(pl.multiple_of(wid * 8, 8), 8)], tail_v)

        def pinit(i, c):
            sup_v[pl.ds(i * LANES, LANES)] = jnp.full((LANES,), POISON, jnp.int32)
            return c

        lax.fori_loop(0, NSUP * SUPCAP // LANES, pinit, 0)

        def piece_body(k, cnts):
            off = pl.multiple_of(wid * BATCH + k * PIECE, PIECE)
            pltpu.sync_copy(xT_hbm.at[pl.ds(off, PIECE)], idx_v)

            def group(i, cnts):
                v16 = idx_v[pl.ds(i * LANES, LANES)]
                p16 = v16 * 16384 + (k * PIECE + i * LANES + iota)
                s16 = ((v16 >> 12) * 21846) >> 16
                new = []
                for s in range(NSUP):
                    m = s16 == s
                    plsc.store_compressed(
                        sup_v.at[pl.ds(s * SUPCAP + cnts[s], LANES)], p16, mask=m)
                    new.append(cnts[s] + plsc.all_reduce_population_count(m)[0])
                return tuple(new)

            return lax.fori_loop(0, PIECE // LANES, group, cnts)

        cnts = lax.fori_loop(0, BATCH // PIECE, piece_body, (0,) * NSUP)
        for s in range(NSUP):
            cnt_sm[s] = cnts[s]

        def scan_list(ns, base, lo, hi):
            def ibody(i, c):
                mg_v[pl.ds(i * LANES, LANES)] = jnp.full((LANES,), dump, jnp.int32)
                return c

            lax.fori_loop(0, MCAP // LANES, ibody, 0)
            groups = jnp.minimum((ns + LANES - 1) // LANES, SUPCAP // LANES)

            def sbody(j, cnt):
                p16 = sup_v[pl.ds(base + j * LANES, LANES)]
                v16 = p16 >> 14
                m = (v16 >= lo) & (v16 < hi)
                dst16 = wid * PADB + (p16 & (16384 - 1))
                plsc.store_compressed(mg_v.at[pl.ds(cnt, LANES)], dst16, mask=m)
                plsc.store_compressed(mv_v.at[pl.ds(cnt, LANES)], v16, mask=m)
                return cnt + plsc.all_reduce_population_count(m)[0]

            return lax.fori_loop(0, groups, sbody, 0)

        def extract_half(n, lo, h):
            groups = jnp.minimum((n + LANES - 1) // LANES, MCAP // LANES)

            def ebody(j, c):
                vl = mv_v[pl.ds(j * LANES, LANES)] - lo
                m = (vl >> 10) == h
                col = vl & (HW - 1)
                r16 = j * LANES + iota
                for e in range(EMB_DIM):
                    e16 = jnp.full((LANES,), e, jnp.int32)
                    vals = plsc.load_gather(slab_v, [e16, col], mask=m)
                    plsc.store_scatter(staged_v, [r16, e16], vals, mask=m)
                return c

            lax.fori_loop(0, groups, ebody, 0)

        def extract_tail(n):
            groups = jnp.minimum((n + LANES - 1) // LANES, MCAP // LANES)

            def ebody(j, c):
                vl = mv_v[pl.ds(j * LANES, LANES)] - TAILLO
                r16 = j * LANES + iota
                for e in range(EMB_DIM):
                    flat = e * TAILW + vl
                    vals = plsc.load_gather(tail_v, [flat >> 7, flat & 127])
                    plsc.store_scatter(
                        staged_v, [r16, jnp.full((LANES,), e, jnp.int32)], vals)
                return c

            lax.fori_loop(0, groups, ebody, 0)

        def flush():
            for j in range(MCAP // 128):
                for k in range(128 // LANES):
                    mg2_v[j, pl.ds(k * LANES, LANES)] = (
                        mg_v[pl.ds(j * 128 + k * LANES, LANES)])
            copies = [
                pltpu.async_copy(staged_v.at[pl.ds(j * 128, 128)],
                                 out_hbm.at[mg2_v.at[j]], ssem)
                for j in range(MCAP // 128)
            ]
            for cp in copies:
                cp.wait()

        def main_chunk(c, carry):
            c6 = c // 6
            lo = pl.multiple_of(c * CW, CW)
            ns = cnt_sm[c6]
            cp = pltpu.async_copy(
                tabT_hbm.at[pl.ds(row0, EMB_DIM), pl.ds(lo, HW)], slab_v, dsem)
            n = scan_list(ns, c6 * SUPCAP, lo, lo + CW)
            cp.wait()
            extract_half(n, lo, 0)
            cp = pltpu.async_copy(
                tabT_hbm.at[pl.ds(row0, EMB_DIM),
                            pl.ds(pl.multiple_of(lo + HW, HW), HW)],
                slab_v, dsem)
            cp.wait()
            extract_half(n, lo, 1)
            flush()
            return carry

        lax.fori_loop(0, NMAIN, main_chunk, 0)

        lo = NMAIN * CW
        cp = pltpu.async_copy(
            tabT_hbm.at[pl.ds(row0, EMB_DIM), pl.ds(lo, HW)], slab_v, dsem)
        n = scan_list(cnt_sm[8], 8 * SUPCAP, lo, TAILLO)
        cp.wait()
        extract_half(n, lo, 0)
        cp = pltpu.async_copy(
            tabT_hbm.at[pl.ds(row0, EMB_DIM), pl.ds(lo + HW, LASTW - HW)],
            slab_v.at[:, pl.ds(0, LASTW - HW)], dsem)
        cp.wait()
        extract_half(n, lo, 1)
        flush()

        n = scan_list(cnt_sm[8], 8 * SUPCAP, TAILLO, VOCAB)
        extract_tail(n)
        flush()


_sc_lookup = functools.partial(
    pl.kernel,
    mesh=plsc.VectorSubcoreMesh(core_axis_name="c", subcore_axis_name="s"),
    out_type=jax.ShapeDtypeStruct((OUTROWS, 128), jnp.float32),
    scratch_types=[
        pltpu.VMEM((PIECE,), jnp.int32),
        pltpu.VMEM((EMB_DIM, HW), jnp.float32),
        pltpu.VMEM((8, 128), jnp.float32),
        pltpu.VMEM((MGCAP,), jnp.int32),
        pltpu.VMEM((MCAP // 128, 128), jnp.int32),
        pltpu.VMEM((MGCAP,), jnp.int32),
        pltpu.VMEM((MCAP, 128), jnp.float32),
        pltpu.VMEM((NSUP * SUPCAP,), jnp.int32),
        pltpu.SMEM((16,), jnp.int32),
        pltpu.SemaphoreType.DMA,
        pltpu.SemaphoreType.DMA,
    ],
    compiler_params=pltpu.CompilerParams(
        use_tc_tiling_on_sc=True, needs_layout_passes=False
    ),
)(_sc_body)


BB = 1024


def _tc_proj_body(c_ref, w_ref, b_ref, g_ref, be_ref, o_ref):
    h = jnp.zeros((BB, OUT_DIM), jnp.float32)
    for f in range(NUM_FIELDS):
        h = h + jnp.dot(c_ref[f, :, :EMB_DIM], w_ref[f],
                        preferred_element_type=jnp.float32)
    h = jnp.maximum(h + b_ref[...], 0.0)
    mean = jnp.mean(h, axis=1, keepdims=True)
    cen = h - mean
    var = jnp.mean(cen * cen, axis=1, keepdims=True)
    o_ref[...] = cen * lax.rsqrt(var + EPS) * g_ref[...] + be_ref[...]


def _tc_proj(rows3, W3, b, gamma, beta):
    return pl.pallas_call(
        _tc_proj_body,
        grid=(BATCH // BB,),
        in_specs=[
            pl.BlockSpec((NUM_FIELDS, BB, 128), lambda i: (0, i, 0)),
            pl.BlockSpec((NUM_FIELDS, EMB_DIM, OUT_DIM), lambda i: (0, 0, 0)),
            pl.BlockSpec((1, OUT_DIM), lambda i: (0, 0)),
            pl.BlockSpec((1, OUT_DIM), lambda i: (0, 0)),
            pl.BlockSpec((1, OUT_DIM), lambda i: (0, 0)),
        ],
        out_specs=pl.BlockSpec((BB, OUT_DIM), lambda i: (i, 0)),
        out_shape=jax.ShapeDtypeStruct((BATCH, OUT_DIM), jnp.float32),
        compiler_params=pltpu.CompilerParams(
            dimension_semantics=("arbitrary",),
        ),
    )(rows3, W3, b, gamma, beta)


def kernel(x, tables, W, b, gamma, beta):
    xT_flat = x.astype(jnp.int32).T.reshape(NUM_FIELDS * BATCH)
    tabT = tables.transpose(0, 2, 1).reshape(TROWS, VOCAB)
    tail = tabT[:, TAILLO:VOCAB].reshape(TROWS * TAILW // 128, 128)

    rows = _sc_lookup(xT_flat, tabT, tail)
    rows3 = rows.reshape(NUM_FIELDS, PADB, 128)
    return _tc_proj(
        rows3,
        W.reshape(NUM_FIELDS, EMB_DIM, OUT_DIM),
        b.reshape(1, OUT_DIM),
        gamma.reshape(1, OUT_DIM),
        beta.reshape(1, OUT_DIM),
    )
